# baseline (speedup 1.0000x reference)
import functools

import jax
import jax.numpy as jnp
from jax import lax
from jax.experimental import pallas as pl
from jax.experimental.pallas import tpu as pltpu
from jax.experimental.pallas import tpu_sc as plsc

N_IDS = 425984
NC = 2
NS = 16
NW = NC * NS
B_PER_W = N_IDS // NW
N_CHUNKS = 4
CH = B_PER_W // N_CHUNKS

_mesh = plsc.VectorSubcoreMesh(core_axis_name="c", subcore_axis_name="s")


@functools.partial(
    pl.kernel,
    mesh=_mesh,
    out_type=jax.ShapeDtypeStruct((N_IDS,), jnp.float32),
    scratch_types=[
        pltpu.VMEM((B_PER_W,), jnp.int32),
        pltpu.VMEM((B_PER_W,), jnp.float32),
        pltpu.SemaphoreType.DMA,
        pltpu.SemaphoreType.DMA,
        pltpu.SemaphoreType.DMA,
        pltpu.SemaphoreType.DMA,
        pltpu.SemaphoreType.DMA,
        pltpu.SemaphoreType.DMA,
    ],
)
def _gather_kernel(ids_hbm, table_hbm, out_hbm, idx_v, rows_v, s_ld,
                   g0, g1, g2, g3, s_wb):
    wid = lax.axis_index("s") * NC + lax.axis_index("c")
    base = wid * B_PER_W
    gsem = [g0, g1, g2, g3]

    def idx_chunk(c):
        return idx_v.at[pl.ds(c * CH, CH)]

    def row_chunk(c):
        return rows_v.at[pl.ds(c * CH, CH)]

    def hbm_chunk(ref, c):
        return ref.at[pl.ds(base + c * CH, CH)]

    pltpu.async_copy(hbm_chunk(ids_hbm, 0), idx_chunk(0), s_ld)
    for c in range(N_CHUNKS):
        pltpu.make_async_copy(hbm_chunk(ids_hbm, c), idx_chunk(c), s_ld).wait()
        if c + 1 < N_CHUNKS:
            pltpu.async_copy(hbm_chunk(ids_hbm, c + 1), idx_chunk(c + 1), s_ld)
        pltpu.async_copy(table_hbm.at[idx_chunk(c)], row_chunk(c), gsem[c])
        if c >= 1:
            pltpu.make_async_copy(table_hbm.at[idx_chunk(c - 1)],
                                  row_chunk(c - 1), gsem[c - 1]).wait()
            pltpu.async_copy(row_chunk(c - 1), hbm_chunk(out_hbm, c - 1), s_wb)
    c = N_CHUNKS - 1
    pltpu.make_async_copy(table_hbm.at[idx_chunk(c)], row_chunk(c),
                          gsem[c]).wait()
    pltpu.async_copy(row_chunk(c), hbm_chunk(out_hbm, c), s_wb)
    pltpu.make_async_copy(out_hbm.at[pl.ds(base, B_PER_W)], rows_v, s_wb).wait()


def kernel(ids, kernel):
    gathered = _gather_kernel(ids, kernel)
    return gathered.reshape(N_IDS, 1)


# speedup vs baseline: 1.0143x; 1.0143x over previous
import functools

import jax
import jax.numpy as jnp
from jax import lax
from jax.experimental import pallas as pl
from jax.experimental.pallas import tpu as pltpu
from jax.experimental.pallas import tpu_sc as plsc

N_IDS = 425984
NC = 2
NS = 16
NW = NC * NS
B_PER_W = N_IDS // NW

_mesh = plsc.VectorSubcoreMesh(core_axis_name="c", subcore_axis_name="s")


@functools.partial(
    pl.kernel,
    mesh=_mesh,
    out_type=jax.ShapeDtypeStruct((N_IDS,), jnp.float32),
    scratch_types=[
        pltpu.VMEM((B_PER_W,), jnp.int32),
        pltpu.VMEM((B_PER_W,), jnp.float32),
        pltpu.SemaphoreType.DMA,
    ],
)
def _gather_kernel(ids_hbm, table_hbm, out_hbm, idx_v, rows_v, sem):
    wid = lax.axis_index("s") * NC + lax.axis_index("c")
    base = wid * B_PER_W
    pltpu.sync_copy(ids_hbm.at[pl.ds(base, B_PER_W)], idx_v)
    pltpu.async_copy(table_hbm.at[idx_v], rows_v, sem).wait()
    pltpu.sync_copy(rows_v, out_hbm.at[pl.ds(base, B_PER_W)])


def kernel(ids, kernel):
    gathered = _gather_kernel(ids, kernel)
    return gathered.reshape(N_IDS, 1)
